# baseline (device time: 91661 ns/iter reference)
import os

import jax
import jax.numpy as jnp
from jax import lax
from jax.experimental import pallas as pl
from jax.experimental.pallas import tpu as pltpu

NOCOMM = bool(int(os.environ.get("NOCOMM", "0")))
N_CHUNK = 32
RING = 4
GATHER_UNROLL = 8


def _fused(idx, scale, E):
    t = idx.shape[0]
    _, d = E.shape
    half = t // 2
    rows = half // N_CHUNK

    def body(
        idx_ref, scale_ref, E_ref, out_ref, fring, ybuf, gsems, ysend, yrecv, xsend, xrecv
    ):
        my_x = lax.axis_index("x")
        my_y = lax.axis_index("y")
        my_z = lax.axis_index("z")
        ynbr = (my_x, 1 - my_y, my_z)
        xnbr = (1 - my_x, my_y, my_z)
        my_off = my_x * half

        if not NOCOMM:
            barrier = pltpu.get_barrier_semaphore()
            for nbr in (ynbr, xnbr):
                pl.semaphore_signal(
                    barrier, inc=1, device_id=nbr, device_id_type=pl.DeviceIdType.MESH
                )
            pl.semaphore_wait(barrier, 2)

        def issue_segment(s):
            slot = s % RING
            base = s * rows

            def b(k, _):
                for u in range(GATHER_UNROLL):
                    j = k * GATHER_UNROLL + u
                    pltpu.make_async_copy(
                        E_ref.at[pl.ds(idx_ref[my_off + base + j], 1), :],
                        fring.at[slot].at[pl.ds(j, 1), :],
                        gsems.at[slot],
                    ).start()
                return 0

            lax.fori_loop(0, rows // GATHER_UNROLL, b, 0)

        def convert(cs):
            pltpu.make_async_copy(
                E_ref.at[pl.ds(0, rows), :],
                fring.at[cs % RING],
                gsems.at[cs % RING],
            ).wait()
            g0 = my_off + cs * rows
            out_ref[pl.ds(g0, rows), :] = (
                fring[cs % RING] * scale_ref[pl.ds(g0, rows), :]
            ).astype(jnp.bfloat16)

        y_rdmas = [None] * N_CHUNK
        x_rdmas = [None] * N_CHUNK

        def y_send(c):
            rd = pltpu.make_async_remote_copy(
                src_ref=out_ref.at[pl.ds(my_off + c * rows, rows), :],
                dst_ref=ybuf.at[pl.ds(c * rows, rows), :],
                send_sem=ysend.at[c],
                recv_sem=yrecv.at[c],
                device_id=ynbr,
                device_id_type=pl.DeviceIdType.MESH,
            )
            rd.start()
            y_rdmas[c] = rd

        def y_process(c):
            y_rdmas[c].wait()
            g0 = my_off + c * rows
            out_ref[pl.ds(g0, rows), :] = (
                out_ref[pl.ds(g0, rows), :] + ybuf[pl.ds(c * rows, rows), :]
            )
            rd = pltpu.make_async_remote_copy(
                src_ref=out_ref.at[pl.ds(g0, rows), :],
                dst_ref=out_ref.at[pl.ds(g0, rows), :],
                send_sem=xsend.at[c],
                recv_sem=xrecv.at[c],
                device_id=xnbr,
                device_id_type=pl.DeviceIdType.MESH,
            )
            rd.start()
            x_rdmas[c] = rd

        for s in range(N_CHUNK):
            issue_segment(s)
            if s >= 1:
                convert(s - 1)
                if not NOCOMM:
                    y_send(s - 1)
            if s >= 2 and not NOCOMM:
                y_process(s - 2)

        convert(N_CHUNK - 1)
        if not NOCOMM:
            y_send(N_CHUNK - 1)
            for c in range(N_CHUNK - 2, N_CHUNK):
                y_process(c)
            for c in range(N_CHUNK):
                x_rdmas[c].wait_recv()
            for c in range(N_CHUNK):
                x_rdmas[c].wait_send()

    return pl.pallas_call(
        body,
        out_shape=jax.ShapeDtypeStruct((t, d), jnp.bfloat16),
        in_specs=[
            pl.BlockSpec(memory_space=pltpu.SMEM),
            pl.BlockSpec(memory_space=pltpu.VMEM),
            pl.BlockSpec(memory_space=pltpu.MemorySpace.HBM),
        ],
        out_specs=pl.BlockSpec(memory_space=pltpu.VMEM),
        scratch_shapes=[
            pltpu.VMEM((RING, rows, d), jnp.float32),
            pltpu.VMEM((half, d), jnp.bfloat16),
            pltpu.SemaphoreType.DMA((RING,)),
            pltpu.SemaphoreType.DMA((N_CHUNK,)),
            pltpu.SemaphoreType.DMA((N_CHUNK,)),
            pltpu.SemaphoreType.DMA((N_CHUNK,)),
            pltpu.SemaphoreType.DMA((N_CHUNK,)),
        ],
        compiler_params=pltpu.CompilerParams(
            collective_id=None if NOCOMM else 0,
            vmem_limit_bytes=56 * 1024 * 1024,
        ),
    )(idx, scale, E)


def kernel(ids, E):
    v_per = E.shape[0]
    my_y = lax.axis_index("y")
    local = ids - my_y * v_per
    mask = (local >= 0) & (local < v_per)
    idx = jnp.clip(local, 0, v_per - 1)
    if os.environ.get("GIDX0"):
        idx = jnp.zeros_like(idx)
    scale = mask.astype(jnp.float32)[:, None]
    return _fused(idx, scale, E)
